# baseline (speedup 1.0000x reference)
import functools

import jax
import jax.numpy as jnp
from jax import lax
from jax.experimental import pallas as pl
from jax.experimental.pallas import tpu as pltpu
from jax.experimental.pallas import tpu_sc as plsc

BATCH = 1024
N_SURF = 200
STATE = 128
NUM_CORES = 2
NUM_SUBCORES = 16
NW = NUM_CORES * NUM_SUBCORES
BPW = 8
N_ACTIVE = N_SURF // BPW
BB = 128


def _sc_gather(table, idx):
    mesh = plsc.VectorSubcoreMesh(core_axis_name="c", subcore_axis_name="s")

    @functools.partial(
        pl.kernel,
        mesh=mesh,
        out_type=jax.ShapeDtypeStruct((N_SURF, STATE), jnp.float32),
        scratch_types=[
            pltpu.VMEM((BPW,), jnp.int32),
            pltpu.VMEM((BPW, STATE), jnp.float32),
            pltpu.SemaphoreType.DMA,
        ],
    )
    def k(table_hbm, idx_hbm, out_hbm, idx_v, rows_v, sem):
        wid = lax.axis_index("s") * NUM_CORES + lax.axis_index("c")

        @pl.when(wid < N_ACTIVE)
        def _():
            base = wid * BPW
            pltpu.sync_copy(idx_hbm.at[pl.ds(base, BPW)], idx_v)
            pltpu.async_copy(table_hbm.at[idx_v], rows_v, sem).wait()
            pltpu.sync_copy(rows_v, out_hbm.at[pl.ds(base, BPW)])

    return k(table, idx)


def _add_body(pe_ref, states_ref, out_ref):
    out_ref[...] = states_ref[...] + pe_ref[...]


def _tc_add(states, pe):
    return pl.pallas_call(
        _add_body,
        grid=(BATCH // BB,),
        in_specs=[
            pl.BlockSpec((1, N_SURF, STATE), lambda i: (0, 0, 0)),
            pl.BlockSpec((BB, N_SURF, STATE), lambda i: (i, 0, 0)),
        ],
        out_specs=pl.BlockSpec((BB, N_SURF, STATE), lambda i: (i, 0, 0)),
        out_shape=jax.ShapeDtypeStruct((BATCH, N_SURF, STATE), jnp.float32),
    )(pe[None], states)


def kernel(states, surface_indices, pos_embedding):
    pe = _sc_gather(pos_embedding, surface_indices)
    return _tc_add(states, pe)


# speedup vs baseline: 1.0047x; 1.0047x over previous
import functools

import jax
import jax.numpy as jnp
from jax import lax
from jax.experimental import pallas as pl
from jax.experimental.pallas import tpu as pltpu
from jax.experimental.pallas import tpu_sc as plsc

BATCH = 1024
N_SURF = 200
STATE = 128
NUM_CORES = 2
NUM_SUBCORES = 16
NW = NUM_CORES * NUM_SUBCORES
BPW = 8
N_ACTIVE = N_SURF // BPW
BB = 128


def _sc_gather(table, idx):
    mesh = plsc.VectorSubcoreMesh(
        core_axis_name="c", subcore_axis_name="s", num_cores=1
    )

    @functools.partial(
        pl.kernel,
        mesh=mesh,
        out_type=jax.ShapeDtypeStruct((N_SURF, STATE), jnp.float32),
        scratch_types=[
            pltpu.VMEM((BPW,), jnp.int32),
            pltpu.VMEM((BPW, STATE), jnp.float32),
            pltpu.SemaphoreType.DMA,
        ],
    )
    def k(table_hbm, idx_hbm, out_hbm, idx_v, rows_v, sem):
        sid = lax.axis_index("s")
        for job in (sid, sid + NUM_SUBCORES):

            @pl.when(job < N_ACTIVE)
            def _():
                base = job * BPW
                pltpu.sync_copy(idx_hbm.at[pl.ds(base, BPW)], idx_v)
                pltpu.async_copy(table_hbm.at[idx_v], rows_v, sem).wait()
                pltpu.sync_copy(rows_v, out_hbm.at[pl.ds(base, BPW)])

    return k(table, idx)


def _add_body(pe_ref, states_ref, out_ref):
    out_ref[...] = states_ref[...] + pe_ref[...]


def _tc_add(states, pe):
    return pl.pallas_call(
        _add_body,
        grid=(BATCH // BB,),
        in_specs=[
            pl.BlockSpec((1, N_SURF, STATE), lambda i: (0, 0, 0)),
            pl.BlockSpec((BB, N_SURF, STATE), lambda i: (i, 0, 0)),
        ],
        out_specs=pl.BlockSpec((BB, N_SURF, STATE), lambda i: (i, 0, 0)),
        out_shape=jax.ShapeDtypeStruct((BATCH, N_SURF, STATE), jnp.float32),
    )(pe[None], states)


def kernel(states, surface_indices, pos_embedding):
    pe = _sc_gather(pos_embedding, surface_indices)
    return _tc_add(states, pe)
